# baseline (speedup 1.0000x reference)
import numpy as np
import jax
import jax.numpy as jnp
from jax.experimental import pallas as pl
from jax.experimental.pallas import tpu as pltpu

N_BINS = 15
_BOUNDS = np.linspace(0.0, 1.0, N_BINS + 1)
_LOWERS = _BOUNDS[:-1].astype(np.float32).reshape(1, N_BINS)
_UPPERS = _BOUNDS[1:].astype(np.float32).reshape(1, N_BINS)

_ROWS_PER_BLOCK = 1000


def _ece_body(x_ref, lab_ref, lo_ref, up_ref, out_ref, acc_ref, *, n_total, n_blocks):
    i = pl.program_id(0)

    @pl.when(i == 0)
    def _init():
        acc_ref[...] = jnp.zeros_like(acc_ref)

    x = x_ref[...]
    rowmax = jnp.max(x, axis=1, keepdims=True)
    sumexp = jnp.sum(jnp.exp(x - rowmax), axis=1, keepdims=True)
    conf = 1.0 / sumexp

    col = jax.lax.broadcasted_iota(jnp.int32, x.shape, 1)
    pred = jnp.min(jnp.where(x == rowmax, col, x.shape[1]),
                   axis=1, keepdims=True)
    acc = (pred == lab_ref[...]).astype(jnp.float32)

    lo = lo_ref[...]
    up = up_ref[...]
    masks = ((conf > lo) & (conf <= up)).astype(jnp.float32)

    cnt_p = jnp.sum(masks, axis=0, keepdims=True)
    conf_p = jnp.sum(masks * conf, axis=0, keepdims=True)
    acc_p = jnp.sum(masks * acc, axis=0, keepdims=True)
    acc_ref[...] += jnp.concatenate([cnt_p, conf_p, acc_p], axis=0)

    @pl.when(i == n_blocks - 1)
    def _finish():
        cnt = acc_ref[0:1, :]
        csum = acc_ref[1:2, :]
        asum = acc_ref[2:3, :]
        denom = jnp.maximum(cnt, 1.0)
        contrib = jnp.abs(csum / denom - asum / denom) * (cnt / n_total)
        out_ref[...] = jnp.sum(jnp.where(cnt > 0, contrib, 0.0),
                               axis=1, keepdims=True)


def kernel(logits, labels):
    n, c = logits.shape
    r = _ROWS_PER_BLOCK
    n_blocks = n // r
    labels2d = labels.astype(jnp.int32).reshape(n, 1)

    import functools
    body = functools.partial(_ece_body, n_total=float(n), n_blocks=n_blocks)
    ece = pl.pallas_call(
        body,
        grid=(n_blocks,),
        in_specs=[
            pl.BlockSpec((r, c), lambda i: (i, 0)),
            pl.BlockSpec((r, 1), lambda i: (i, 0)),
            pl.BlockSpec((1, N_BINS), lambda i: (0, 0)),
            pl.BlockSpec((1, N_BINS), lambda i: (0, 0)),
        ],
        out_specs=pl.BlockSpec((1, 1), lambda i: (0, 0)),
        out_shape=jax.ShapeDtypeStruct((1, 1), jnp.float32),
        scratch_shapes=[pltpu.VMEM((3, N_BINS), jnp.float32)],
    )(logits, labels2d, jnp.asarray(_LOWERS), jnp.asarray(_UPPERS))
    return ece.reshape(1)


# speedup vs baseline: 1.0424x; 1.0424x over previous
import numpy as np
import jax
import jax.numpy as jnp
from jax.experimental import pallas as pl
from jax.experimental.pallas import tpu as pltpu

N_BINS = 15
_BOUNDS = np.linspace(0.0, 1.0, N_BINS + 1)
_LOWERS = _BOUNDS[:-1].astype(np.float32).reshape(1, N_BINS)
_UPPERS = _BOUNDS[1:].astype(np.float32).reshape(1, N_BINS)

_ROWS_PER_BLOCK = 2000


def _ece_body(x_ref, lab_ref, lo_ref, up_ref, out_ref, acc_ref, *, n_total, n_blocks):
    i = pl.program_id(0)

    @pl.when(i == 0)
    def _init():
        acc_ref[...] = jnp.zeros_like(acc_ref)

    x = x_ref[...]
    rowmax = jnp.max(x, axis=1, keepdims=True)
    sumexp = jnp.sum(jnp.exp(x - rowmax), axis=1, keepdims=True)
    conf = 1.0 / sumexp

    col = jax.lax.broadcasted_iota(jnp.int32, x.shape, 1)
    pred = jnp.min(jnp.where(x == rowmax, col, x.shape[1]),
                   axis=1, keepdims=True)
    acc = (pred == lab_ref[...]).astype(jnp.float32)

    lo = lo_ref[...]
    up = up_ref[...]
    masks = ((conf > lo) & (conf <= up)).astype(jnp.float32)

    cnt_p = jnp.sum(masks, axis=0, keepdims=True)
    conf_p = jnp.sum(masks * conf, axis=0, keepdims=True)
    acc_p = jnp.sum(masks * acc, axis=0, keepdims=True)
    acc_ref[...] += jnp.concatenate([cnt_p, conf_p, acc_p], axis=0)

    @pl.when(i == n_blocks - 1)
    def _finish():
        cnt = acc_ref[0:1, :]
        csum = acc_ref[1:2, :]
        asum = acc_ref[2:3, :]
        denom = jnp.maximum(cnt, 1.0)
        contrib = jnp.abs(csum / denom - asum / denom) * (cnt / n_total)
        out_ref[...] = jnp.sum(jnp.where(cnt > 0, contrib, 0.0),
                               axis=1, keepdims=True)


def kernel(logits, labels):
    n, c = logits.shape
    r = _ROWS_PER_BLOCK
    n_blocks = n // r
    labels2d = labels.astype(jnp.int32).reshape(n, 1)

    import functools
    body = functools.partial(_ece_body, n_total=float(n), n_blocks=n_blocks)
    ece = pl.pallas_call(
        body,
        grid=(n_blocks,),
        in_specs=[
            pl.BlockSpec((r, c), lambda i: (i, 0)),
            pl.BlockSpec((r, 1), lambda i: (i, 0)),
            pl.BlockSpec((1, N_BINS), lambda i: (0, 0)),
            pl.BlockSpec((1, N_BINS), lambda i: (0, 0)),
        ],
        out_specs=pl.BlockSpec((1, 1), lambda i: (0, 0)),
        out_shape=jax.ShapeDtypeStruct((1, 1), jnp.float32),
        scratch_shapes=[pltpu.VMEM((3, N_BINS), jnp.float32)],
    )(logits, labels2d, jnp.asarray(_LOWERS), jnp.asarray(_UPPERS))
    return ece.reshape(1)
